# Initial kernel scaffold; baseline (speedup 1.0000x reference)
import jax
import jax.numpy as jnp
from jax.experimental import pallas as pl


def kernel(x, cache, params):
    raise NotImplementedError("write your pallas kernel here")


# speedup vs baseline: 1.7500x; 1.7500x over previous
import functools

import jax
import jax.numpy as jnp
from jax.experimental import pallas as pl
from jax.experimental.pallas import tpu as pltpu

H = 16
KW = 64
EPS = 1e-5
BS = 256


def _dot_nt(a, b):
    return jax.lax.dot_general(
        a, b, (((1,), (1,)), ((), ())), preferred_element_type=jnp.float32)


def _dot_nn(a, b):
    return jax.lax.dot_general(
        a, b, (((1,), (0,)), ((), ())), preferred_element_type=jnp.float32)


def _dot_tn(a, b):
    return jax.lax.dot_general(
        a, b, (((0,), (0,)), ((), ())), preferred_element_type=jnp.float32)


def _rowdot(x, w_row, bias):
    return jnp.sum(x * w_row, axis=-1, keepdims=True) + bias


def _ln(x, g, b):
    m = jnp.mean(x, axis=-1, keepdims=True)
    v = jnp.mean((x - m) ** 2, axis=-1, keepdims=True)
    return (x - m) * jax.lax.rsqrt(v + EPS) * g + b


def _softmax(s):
    s = s - jnp.max(s, axis=-1, keepdims=True)
    e = jnp.exp(s)
    return e / jnp.sum(e, axis=-1, keepdims=True)


def _read_qkv_body(feedback, dc_scale,
                   h_ref, cache_ref, rq_w, rq_b, ro_w, ro_b, rg_w, rg_b,
                   ln_g, ln_b, qkv_w, qkv_b, fb_w, fb_b,
                   henh_ref, qkv_ref):
    h = h_ref[0]
    if feedback:
        g = jax.nn.sigmoid(_dot_nt(h, fb_w[...]) + fb_b[...])
        h = h + g * h
    cache = cache_ref[0]
    q = _dot_nt(h, rq_w[...]) + rq_b[...]
    att = _softmax(_dot_nt(q, cache) * dc_scale)
    r = _dot_nn(att, cache)
    gate = jax.nn.sigmoid(_rowdot(h, rg_w[...], rg_b[0, 0]))
    henh = h + gate * (_dot_nt(r, ro_w[...]) + ro_b[...])
    henh_ref[0] = henh
    xn = _ln(henh, ln_g[...], ln_b[...])
    qkv_ref[0] = _dot_nt(xn, qkv_w[...]) + qkv_b[...]


def _make_read_call(B, S, D, N, DC, feedback):
    dc_scale = 1.0 / (DC ** 0.5)
    full = lambda shape: pl.BlockSpec(shape, lambda b, i: (0,) * len(shape))
    grid = (B, S // BS)
    return pl.pallas_call(
        functools.partial(_read_qkv_body, feedback, dc_scale),
        grid=grid,
        in_specs=[
            pl.BlockSpec((1, BS, D), lambda b, i: (b, i, 0)),
            pl.BlockSpec((1, N, DC), lambda b, i: (b, 0, 0)),
            full((DC, D)), full((1, DC)),
            full((D, DC)), full((1, D)),
            full((1, D)), full((1, 1)),
            full((1, D)), full((1, D)),
            full((3 * D, D)), full((1, 3 * D)),
            full((D, D)), full((1, D)),
        ],
        out_specs=[
            pl.BlockSpec((1, BS, D), lambda b, i: (b, i, 0)),
            pl.BlockSpec((1, BS, 3 * D), lambda b, i: (b, i, 0)),
        ],
        out_shape=[
            jax.ShapeDtypeStruct((B, S, D), jnp.float32),
            jax.ShapeDtypeStruct((B, S, 3 * D), jnp.float32),
        ],
        compiler_params=pltpu.CompilerParams(
            dimension_semantics=("parallel", "arbitrary"),
            vmem_limit_bytes=56 * 1024 * 1024,
        ),
        name="read_qkv",
    )


def _attn_body(D, q_ref, k_ref, v_ref, o_ref):
    DH = D // H
    scale = 1.0 / (DH ** 0.5)
    q = q_ref[0]
    k = k_ref[0]
    v = v_ref[0]
    outs = []
    for hh in range(H):
        sl = slice(hh * DH, (hh + 1) * DH)
        s = _dot_nt(q[:, sl], k[:, sl]) * scale
        p = _softmax(s)
        outs.append(_dot_nn(p, v[:, sl]))
    o_ref[0] = jnp.concatenate(outs, axis=1)


def _make_attn_call(B, S, D):
    grid = (B, S // BS)
    return pl.pallas_call(
        functools.partial(_attn_body, D),
        grid=grid,
        in_specs=[
            pl.BlockSpec((1, BS, D), lambda b, i: (b, i, 0)),
            pl.BlockSpec((1, S, D), lambda b, i: (b, 0, 1)),
            pl.BlockSpec((1, S, D), lambda b, i: (b, 0, 2)),
        ],
        out_specs=pl.BlockSpec((1, BS, D), lambda b, i: (b, i, 0)),
        out_shape=jax.ShapeDtypeStruct((B, S, D), jnp.float32),
        compiler_params=pltpu.CompilerParams(
            dimension_semantics=("parallel", "arbitrary"),
            vmem_limit_bytes=56 * 1024 * 1024,
        ),
        name="mha",
    )


def _ffn_body(henh_ref, attn_ref, out_w, out_b, ln1_g, ln1_b,
              ffn1_w, ffn1_b, ffn2_w, ffn2_b, ln2_g, ln2_b, imp_w, imp_b,
              h_ref, imp_ref):
    a = _dot_nt(attn_ref[0], out_w[...]) + out_b[...]
    x = _ln(henh_ref[0] + a, ln1_g[...], ln1_b[...])
    hid = jax.nn.gelu(_dot_nt(x, ffn1_w[...]) + ffn1_b[...])
    f = _dot_nt(hid, ffn2_w[...]) + ffn2_b[...]
    out = _ln(x + f, ln2_g[...], ln2_b[...])
    h_ref[0] = out
    imp_ref[0] = _rowdot(out, imp_w[...], imp_b[0, 0])


def _make_ffn_call(B, S, D):
    full = lambda shape: pl.BlockSpec(shape, lambda b, i: (0,) * len(shape))
    grid = (B, S // BS)
    return pl.pallas_call(
        _ffn_body,
        grid=grid,
        in_specs=[
            pl.BlockSpec((1, BS, D), lambda b, i: (b, i, 0)),
            pl.BlockSpec((1, BS, D), lambda b, i: (b, i, 0)),
            full((D, D)), full((1, D)),
            full((1, D)), full((1, D)),
            full((4 * D, D)), full((1, 4 * D)),
            full((D, 4 * D)), full((1, D)),
            full((1, D)), full((1, D)),
            full((1, D)), full((1, 1)),
        ],
        out_specs=[
            pl.BlockSpec((1, BS, D), lambda b, i: (b, i, 0)),
            pl.BlockSpec((1, BS, 1), lambda b, i: (b, i, 0)),
        ],
        out_shape=[
            jax.ShapeDtypeStruct((B, S, D), jnp.float32),
            jax.ShapeDtypeStruct((B, S, 1), jnp.float32),
        ],
        compiler_params=pltpu.CompilerParams(
            dimension_semantics=("parallel", "arbitrary"),
            vmem_limit_bytes=56 * 1024 * 1024,
        ),
        name="ffn_imp",
    )


def _write_body(S, dc_scale,
                h_ref, impc_ref, impr_ref, cache_ref,
                wk_w, wk_b, wv_w, wv_b, wg_w, wg_b,
                out_ref, delta_acc, tot_acc):
    c = pl.program_id(1)
    nchunks = pl.num_programs(1)

    @pl.when(c == 0)
    def _():
        delta_acc[...] = jnp.zeros_like(delta_acc)
        tot_acc[...] = jnp.zeros_like(tot_acc)

    sel = h_ref[0]
    cache = cache_ref[0]
    keys = _dot_nt(sel, wk_w[...]) + wk_b[...]
    vals = _dot_nt(sel, wv_w[...]) + wv_b[...]
    gate = jax.nn.sigmoid(_rowdot(sel, wg_w[...], wg_b[0, 0]))
    route = _softmax(_dot_nt(keys, cache) * dc_scale)

    imp_all = impr_ref[0]
    imp_c = impc_ref[0]
    ia = jax.lax.broadcasted_iota(jnp.int32, (BS, S), 1)
    ic = jax.lax.broadcasted_iota(jnp.int32, (BS, S), 0) + c * BS
    beats = (imp_all > imp_c) | ((imp_all == imp_c) & (ia < ic))
    rank = jnp.sum(jnp.where(beats, 1.0, 0.0), axis=1, keepdims=True)
    mask = jnp.where(rank < float(KW), 1.0, 0.0)

    w = route * (gate * mask)
    delta_acc[...] += _dot_tn(w, vals)
    tot_acc[...] += _dot_tn(w, jnp.ones((w.shape[0], 1), jnp.float32))

    @pl.when(c == nchunks - 1)
    def _():
        tot = jnp.clip(tot_acc[...], 0.0, 1.0)
        out_ref[0] = cache * (1.0 - tot) + delta_acc[...]


def _make_write_call(B, S, D, N, DC):
    dc_scale = 1.0 / (DC ** 0.5)
    full = lambda shape: pl.BlockSpec(shape, lambda b, i: (0,) * len(shape))
    grid = (B, S // BS)
    return pl.pallas_call(
        functools.partial(_write_body, S, dc_scale),
        grid=grid,
        in_specs=[
            pl.BlockSpec((1, BS, D), lambda b, i: (b, i, 0)),
            pl.BlockSpec((1, BS, 1), lambda b, i: (b, i, 0)),
            pl.BlockSpec((1, 1, S), lambda b, i: (b, 0, 0)),
            pl.BlockSpec((1, N, DC), lambda b, i: (b, 0, 0)),
            full((DC, D)), full((1, DC)),
            full((DC, D)), full((1, DC)),
            full((1, D)), full((1, 1)),
        ],
        out_specs=pl.BlockSpec((1, N, DC), lambda b, i: (b, 0, 0)),
        out_shape=jax.ShapeDtypeStruct((B, N, DC), jnp.float32),
        scratch_shapes=[
            pltpu.VMEM((N, DC), jnp.float32),
            pltpu.VMEM((N, 1), jnp.float32),
        ],
        compiler_params=pltpu.CompilerParams(
            dimension_semantics=("parallel", "arbitrary"),
            vmem_limit_bytes=56 * 1024 * 1024,
        ),
        name="mem_write",
    )


def kernel(x, cache, params):
    B, S, D = x.shape
    N, DC = cache.shape[1], cache.shape[2]
    p = params
    r2 = lambda a: a.reshape(1, -1)

    fb_w = p['fb_w'][:, :D] + p['fb_w'][:, D:]
    read_nofb = _make_read_call(B, S, D, N, DC, False)
    read_fb = _make_read_call(B, S, D, N, DC, True)
    attn_call = _make_attn_call(B, S, D)
    ffn_call = _make_ffn_call(B, S, D)
    write_call = _make_write_call(B, S, D, N, DC)

    read_args = lambda h, c: (
        h, c, p['rq_w'], r2(p['rq_b']), p['ro_w'], r2(p['ro_b']),
        p['rg_w'], r2(p['rg_b']), r2(p['ln1_g']), r2(p['ln1_b']),
        p['qkv_w'], r2(p['qkv_b']), fb_w, r2(p['fb_b']))
    ffn_args = lambda henh, attn: (
        henh, attn, p['out_w'], r2(p['out_b']), r2(p['ln1_g']), r2(p['ln1_b']),
        p['ffn1_w'], r2(p['ffn1_b']), p['ffn2_w'], r2(p['ffn2_b']),
        r2(p['ln2_g']), r2(p['ln2_b']), p['imp_w'], r2(p['imp_b']))

    h, c = x, cache
    for it in range(2):
        rd = read_fb if it > 0 else read_nofb
        henh, qkv = rd(*read_args(h, c))
        attn = attn_call(qkv, qkv, qkv)
        h, impc = ffn_call(*ffn_args(henh, attn))
        impr = jnp.transpose(impc, (0, 2, 1))
        c = write_call(h, impc, impr, c,
                       p['wk_w'], r2(p['wk_b']), p['wv_w'], r2(p['wv_b']),
                       p['wg_w'], r2(p['wg_b']))
    return h, c
